# baseline (speedup 1.0000x reference)
import functools

import jax
import jax.numpy as jnp
from jax import lax
from jax.experimental import pallas as pl
from jax.experimental.pallas import tpu as pltpu
from jax.experimental.pallas import tpu_sc as plsc

D = 64
NUM_CORES = 2
NUM_SUBCORES = 16
NW = NUM_CORES * NUM_SUBCORES
LANES = 16

CH = 128
FIRE = 8


def _gather_kernel(e_total):
    e_per_w = e_total // NW
    n_chunk = e_per_w // CH

    @functools.partial(
        pl.kernel,
        mesh=plsc.VectorSubcoreMesh(core_axis_name="c", subcore_axis_name="s"),
        out_type=jax.ShapeDtypeStruct((e_total,), jnp.float32),
        scratch_types=[
            pltpu.VMEM((e_per_w,), jnp.int32),
            pltpu.VMEM((n_chunk, CH), jnp.int32),
            pltpu.VMEM((e_per_w,), jnp.float32),
            pltpu.SemaphoreType.DMA,
        ],
    )
    def k(tbl_hbm, idx_hbm, out_hbm, idx_v, fidx_v, out_v, sem):
        wid = lax.axis_index("s") * NUM_CORES + lax.axis_index("c")
        base = wid * e_per_w

        pltpu.sync_copy(idx_hbm.at[pl.ds(base, e_per_w)], idx_v)

        lane = lax.iota(jnp.int32, LANES)

        def compute_row(r, carry):
            for c in range(CH // LANES):
                v = idx_v[pl.ds(r * CH + c * LANES, LANES)]
                col = lane + (c % 4) * LANES
                fidx_v[r, pl.ds(c * LANES, LANES)] = v * D + col
            return carry

        lax.fori_loop(0, n_chunk, compute_row, 0, unroll=False)

        def fire_group(g, carry):
            copies = []
            for t in range(FIRE):
                r = g * FIRE + t
                copies.append(
                    pltpu.async_copy(
                        tbl_hbm.at[fidx_v.at[r]],
                        out_v.at[pl.ds(r * CH, CH)],
                        sem,
                    )
                )
            for cp in copies:
                cp.wait()
            return carry

        lax.fori_loop(0, n_chunk // FIRE, fire_group, 0, unroll=False)

        pltpu.sync_copy(out_v, out_hbm.at[pl.ds(base, e_per_w)])

    return k


def kernel(self_tensor, indices):
    n, d = self_tensor.shape
    b, d2 = indices.shape
    assert d == D and d2 == D
    tbl = self_tensor.reshape(n * d)
    idx = indices.reshape(b * d)
    out = _gather_kernel(b * d)(tbl, idx)
    return out.reshape(b, d)


# speedup vs baseline: 1.0319x; 1.0319x over previous
import functools

import jax
import jax.numpy as jnp
from jax import lax
from jax.experimental import pallas as pl
from jax.experimental.pallas import tpu as pltpu
from jax.experimental.pallas import tpu_sc as plsc

D = 64
NUM_CORES = 2
NUM_SUBCORES = 16
NW = NUM_CORES * NUM_SUBCORES
LANES = 16


def _gather_kernel(e_total):
    e_per_w = e_total // NW

    @functools.partial(
        pl.kernel,
        mesh=plsc.VectorSubcoreMesh(core_axis_name="c", subcore_axis_name="s"),
        out_type=jax.ShapeDtypeStruct((e_total,), jnp.float32),
        scratch_types=[
            pltpu.VMEM((e_per_w,), jnp.int32),
            pltpu.VMEM((e_per_w,), jnp.float32),
            pltpu.SemaphoreType.DMA,
        ],
    )
    def k(tbl_hbm, idx_hbm, out_hbm, fidx_v, out_v, sem):
        wid = lax.axis_index("s") * NUM_CORES + lax.axis_index("c")
        base = wid * e_per_w

        pltpu.sync_copy(idx_hbm.at[pl.ds(base, e_per_w)], fidx_v)

        lane = lax.iota(jnp.int32, LANES)

        def body(i, carry):
            for c in range(D // LANES):
                off = i * D + c * LANES
                v = fidx_v[pl.ds(off, LANES)]
                fidx_v[pl.ds(off, LANES)] = v * D + (lane + c * LANES)
            return carry

        lax.fori_loop(0, e_per_w // D, body, 0, unroll=False)

        pltpu.async_copy(tbl_hbm.at[fidx_v], out_v, sem).wait()
        pltpu.sync_copy(out_v, out_hbm.at[pl.ds(base, e_per_w)])

    return k


def kernel(self_tensor, indices):
    n, d = self_tensor.shape
    b, d2 = indices.shape
    assert d == D and d2 == D
    tbl = self_tensor.reshape(n * d)
    idx = indices.reshape(b * d)
    out = _gather_kernel(b * d)(tbl, idx)
    return out.reshape(b, d)
